# baseline (speedup 1.0000x reference)
import functools

import jax
import jax.numpy as jnp
from jax import lax
from jax.experimental import pallas as pl
from jax.experimental.pallas import tpu as pltpu
from jax.experimental.pallas import tpu_sc as plsc

_N = 4096
_K = 50
_KP = 56
_D = 64
_DP = 128
_NC = 2
_NS = 16
_NW = _NC * _NS
_ROWS_PER_W = _N // _NW

_mesh = plsc.VectorSubcoreMesh(core_axis_name="c", subcore_axis_name="s")


@functools.partial(
    pl.kernel,
    mesh=_mesh,
    out_type=jax.ShapeDtypeStruct((_N, _KP, _DP), jnp.float32),
    scratch_types=[
        pltpu.VMEM((_ROWS_PER_W, _KP), jnp.int32),
        pltpu.VMEM((_KP, _D), jnp.float32),
        pltpu.VMEM((_KP, _D), jnp.float32),
        pltpu.SemaphoreType.DMA,
        pltpu.SemaphoreType.DMA,
        pltpu.SemaphoreType.DMA,
        pltpu.SemaphoreType.DMA,
    ],
    compiler_params=pltpu.CompilerParams(use_tc_tiling_on_sc=False),
)
def _gather_kernel(idx_hbm, table_hbm, out_hbm, idx_v, rows0, rows1,
                   gsem0, gsem1, ssem0, ssem1):
    wid = lax.axis_index("s") * _NC + lax.axis_index("c")
    row0 = wid * _ROWS_PER_W

    rows = (rows0, rows1)
    gsem = (gsem0, gsem1)
    ssem = (ssem0, ssem1)

    pltpu.sync_copy(idx_hbm.at[pl.ds(row0, _ROWS_PER_W), :], idx_v)

    def gather(k, b):
        pltpu.async_copy(table_hbm.at[idx_v.at[k]], rows[b], gsem[b])

    def wait_gather(b):
        pltpu.make_async_copy(table_hbm.at[idx_v.at[0]], rows[b],
                              gsem[b]).wait()

    def store(k, b):
        pltpu.async_copy(
            rows[b].at[pl.ds(0, _K), :],
            out_hbm.at[row0 + k, pl.ds(0, _K), pl.ds(0, _D)], ssem[b])

    def wait_store(b):
        pltpu.make_async_copy(
            rows[b].at[pl.ds(0, _K), :],
            out_hbm.at[row0, pl.ds(0, _K), pl.ds(0, _D)], ssem[b]).wait()

    gather(0, 0)

    def body(t, carry):
        k = 2 * t

        @pl.when(t > 0)
        def _():
            wait_store(1)
        gather(k + 1, 1)
        wait_gather(0)
        store(k, 0)

        @pl.when(t < _ROWS_PER_W // 2 - 1)
        def _():
            wait_store(0)
            gather(k + 2, 0)
        wait_gather(1)
        store(k + 1, 1)
        return carry

    lax.fori_loop(0, _ROWS_PER_W // 2, body, 0)
    wait_store(0)
    wait_store(1)


def kernel(indices, embedding_table):
    idx_padded = jnp.pad(indices.astype(jnp.int32), ((0, 0), (0, _KP - _K)))
    out_padded = _gather_kernel(idx_padded, embedding_table)
    return lax.slice(out_padded, (0, 0, 0), (_N, _K, _D))


# speedup vs baseline: 1.0063x; 1.0063x over previous
import functools

import jax
import jax.numpy as jnp
from jax import lax
from jax.experimental import pallas as pl
from jax.experimental.pallas import tpu as pltpu
from jax.experimental.pallas import tpu_sc as plsc

_N = 4096
_K = 50
_KP = 56
_D = 64
_DP = 128
_NC = 2
_NS = 16
_NW = _NC * _NS
_ROWS_PER_W = _N // _NW
_IDX_PER_W = _ROWS_PER_W * _KP
_RCHUNK = 16
_CHUNK = _RCHUNK * _KP
_NCHUNK = _ROWS_PER_W // _RCHUNK

_mesh = plsc.VectorSubcoreMesh(core_axis_name="c", subcore_axis_name="s")


@functools.partial(
    pl.kernel,
    mesh=_mesh,
    out_type=jax.ShapeDtypeStruct((_N, _KP, _DP), jnp.float32),
    scratch_types=[
        pltpu.VMEM((_IDX_PER_W,), jnp.int32),
        pltpu.VMEM((_CHUNK, _D), jnp.float32),
        pltpu.VMEM((_CHUNK, _D), jnp.float32),
        pltpu.SemaphoreType.DMA,
        pltpu.SemaphoreType.DMA,
        pltpu.SemaphoreType.DMA,
        pltpu.SemaphoreType.DMA,
    ],
    compiler_params=pltpu.CompilerParams(use_tc_tiling_on_sc=False),
)
def _gather_kernel(idx_hbm, table_hbm, out_hbm, idx_v, rows0, rows1,
                   gsem0, gsem1, ssem0, ssem1):
    wid = lax.axis_index("s") * _NC + lax.axis_index("c")
    row0 = wid * _ROWS_PER_W

    rows = (rows0, rows1)
    gsem = (gsem0, gsem1)
    ssem = (ssem0, ssem1)

    pltpu.sync_copy(idx_hbm.at[wid], idx_v)

    def gather(g, b):
        return pltpu.async_copy(
            table_hbm.at[idx_v.at[pl.ds(g * _CHUNK, _CHUNK)]], rows[b],
            gsem[b])

    def store(g, b):
        last = None
        for k in range(_RCHUNK):
            last = pltpu.async_copy(
                rows[b].at[pl.ds(k * _KP, _K)],
                out_hbm.at[row0 + g * _RCHUNK + k, pl.ds(0, _K),
                           pl.ds(0, _D)],
                ssem[b])
        return last

    stores = [None, None]
    gathers = [None, None]
    gathers[0] = gather(0, 0)
    for g in range(_NCHUNK):
        b = g % 2
        nb = (g + 1) % 2
        if g + 1 < _NCHUNK:
            if g >= 1:
                for _ in range(_RCHUNK):
                    stores[nb].wait()
            gathers[nb] = gather(g + 1, nb)
        gathers[b].wait()
        stores[b] = store(g, b)
    for _ in range(_RCHUNK):
        stores[(_NCHUNK - 2) % 2].wait()
    for _ in range(_RCHUNK):
        stores[(_NCHUNK - 1) % 2].wait()


def kernel(indices, embedding_table):
    idx_padded = jnp.pad(indices.astype(jnp.int32), ((0, 0), (0, _KP - _K)))
    idx_flat = idx_padded.reshape(_NW, _IDX_PER_W)
    out_padded = _gather_kernel(idx_flat, embedding_table)
    return lax.slice(out_padded, (0, 0, 0), (_N, _K, _D))


# speedup vs baseline: 4.1904x; 4.1642x over previous
import functools

import jax
import jax.numpy as jnp
from jax import lax
from jax.experimental import pallas as pl
from jax.experimental.pallas import tpu as pltpu
from jax.experimental.pallas import tpu_sc as plsc

_N = 4096
_K = 50
_B = _N * _K
_D = 64
_NC = 2
_NS = 16
_NW = _NC * _NS
_ROWS_PER_W = _N // _NW
_RCHUNK = 8
_CHUNK = _RCHUNK * _K
_NCHUNK = _ROWS_PER_W // _RCHUNK
_B_PER_W = _ROWS_PER_W * _K

_mesh = plsc.VectorSubcoreMesh(core_axis_name="c", subcore_axis_name="s")


_KP = 56
_DP = 128


@functools.partial(
    pl.kernel,
    mesh=_mesh,
    out_type=jax.ShapeDtypeStruct((_N, _KP, _DP), jnp.float32),
    scratch_types=[
        pltpu.VMEM((_B_PER_W,), jnp.int32),
        pltpu.VMEM((_CHUNK, _D), jnp.float32),
        pltpu.VMEM((_CHUNK, _D), jnp.float32),
        pltpu.VMEM((_CHUNK, _D), jnp.float32),
        pltpu.VMEM((_CHUNK, _D), jnp.float32),
        pltpu.SemaphoreType.DMA,
        pltpu.SemaphoreType.DMA,
        pltpu.SemaphoreType.DMA,
        pltpu.SemaphoreType.DMA,
        pltpu.SemaphoreType.DMA,
        pltpu.SemaphoreType.DMA,
        pltpu.SemaphoreType.DMA,
        pltpu.SemaphoreType.DMA,
    ],
    compiler_params=pltpu.CompilerParams(use_tc_tiling_on_sc=False),
)
def _gather_kernel(idx_hbm, table_hbm, out_hbm, idx_v, rows0, rows1, rows2,
                   rows3, gsem0, gsem1, gsem2, gsem3, ssem0, ssem1, ssem2,
                   ssem3):
    wid = lax.axis_index("s") * _NC + lax.axis_index("c")
    base = wid * _B_PER_W
    row0 = wid * _ROWS_PER_W

    rows = (rows0, rows1, rows2, rows3)
    gsem = (gsem0, gsem1, gsem2, gsem3)
    ssem = (ssem0, ssem1, ssem2, ssem3)

    pltpu.sync_copy(idx_hbm.at[pl.ds(base, _B_PER_W)], idx_v)

    def gather(g, b):
        return pltpu.async_copy(
            table_hbm.at[idx_v.at[pl.ds(g * _CHUNK, _CHUNK)]], rows[b],
            gsem[b])

    def store(g, b):
        last = None
        for k in range(_RCHUNK):
            last = pltpu.async_copy(
                rows[b].at[pl.ds(k * _K, _K)],
                out_hbm.at[row0 + g * _RCHUNK + k, pl.ds(0, _K),
                           pl.ds(0, _D)],
                ssem[b])
        return last

    stores = [None] * 4
    gathers = [None] * 4
    gathers[0] = gather(0, 0)
    gathers[1] = gather(1, 1)
    for g in range(_NCHUNK):
        b = g % 4
        gathers[b].wait()
        stores[b] = store(g, b)
        if g + 2 < _NCHUNK:
            nb = (g + 2) % 4
            if g >= 2:
                for _ in range(_RCHUNK):
                    stores[nb].wait()
            gathers[nb] = gather(g + 2, nb)
    for d in range(4):
        for _ in range(_RCHUNK):
            stores[(_NCHUNK - 4 + d) % 4].wait()


def kernel(indices, embedding_table):
    flat = jnp.maximum(indices.reshape(-1).astype(jnp.int32), 0)
    out_padded = _gather_kernel(flat, embedding_table)
    return lax.slice(out_padded, (0, 0, 0), (_N, _K, _D))


# speedup vs baseline: 4.2279x; 1.0090x over previous
import functools

import jax
import jax.numpy as jnp
from jax import lax
from jax.experimental import pallas as pl
from jax.experimental.pallas import tpu as pltpu
from jax.experimental.pallas import tpu_sc as plsc

_N = 4096
_K = 50
_B = _N * _K
_D = 64
_NC = 2
_NS = 16
_NW = _NC * _NS
_ROWS_PER_W = _N // _NW
_RCHUNK = 16
_CHUNK = _RCHUNK * _K
_NCHUNK = _ROWS_PER_W // _RCHUNK
_B_PER_W = _ROWS_PER_W * _K

_mesh = plsc.VectorSubcoreMesh(core_axis_name="c", subcore_axis_name="s")


_KP = 56
_DP = 128


@functools.partial(
    pl.kernel,
    mesh=_mesh,
    out_type=jax.ShapeDtypeStruct((_N, _KP, _DP), jnp.float32),
    scratch_types=[
        pltpu.VMEM((_B_PER_W,), jnp.int32),
        pltpu.VMEM((_CHUNK, _D), jnp.float32),
        pltpu.VMEM((_CHUNK, _D), jnp.float32),
        pltpu.SemaphoreType.DMA,
        pltpu.SemaphoreType.DMA,
        pltpu.SemaphoreType.DMA,
        pltpu.SemaphoreType.DMA,
    ],
    compiler_params=pltpu.CompilerParams(use_tc_tiling_on_sc=False),
)
def _gather_kernel(idx_hbm, table_hbm, out_hbm, idx_v, rows0, rows1,
                   gsem0, gsem1, ssem0, ssem1):
    wid = lax.axis_index("s") * _NC + lax.axis_index("c")
    base = wid * _B_PER_W
    row0 = wid * _ROWS_PER_W

    rows = (rows0, rows1)
    gsem = (gsem0, gsem1)
    ssem = (ssem0, ssem1)

    pltpu.sync_copy(idx_hbm.at[pl.ds(base, _B_PER_W)], idx_v)

    def gather(g, b):
        return pltpu.async_copy(
            table_hbm.at[idx_v.at[pl.ds(g * _CHUNK, _CHUNK)]], rows[b],
            gsem[b])

    def store(g, b):
        last = None
        for k in range(_RCHUNK):
            last = pltpu.async_copy(
                rows[b].at[pl.ds(k * _K, _K)],
                out_hbm.at[row0 + g * _RCHUNK + k, pl.ds(0, _K),
                           pl.ds(0, _D)],
                ssem[b])
        return last

    stores = [None, None]
    gathers = [None, None]
    gathers[0] = gather(0, 0)
    for g in range(_NCHUNK):
        b = g % 2
        nb = (g + 1) % 2
        if g + 1 < _NCHUNK:
            if g >= 1:
                for _ in range(_RCHUNK):
                    stores[nb].wait()
            gathers[nb] = gather(g + 1, nb)
        gathers[b].wait()
        stores[b] = store(g, b)
    for _ in range(_RCHUNK):
        stores[(_NCHUNK - 2) % 2].wait()
    for _ in range(_RCHUNK):
        stores[(_NCHUNK - 1) % 2].wait()


def kernel(indices, embedding_table):
    flat = jnp.maximum(indices.reshape(-1).astype(jnp.int32), 0)
    out_padded = _gather_kernel(flat, embedding_table)
    return lax.slice(out_padded, (0, 0, 0), (_N, _K, _D))
